# Initial kernel scaffold; baseline (speedup 1.0000x reference)
import jax
import jax.numpy as jnp
from jax.experimental import pallas as pl


def kernel(x, table):
    raise NotImplementedError("write your pallas kernel here")


# speedup vs baseline: 2.1673x; 2.1673x over previous
import jax
import jax.numpy as jnp
from jax import lax
from jax.experimental import pallas as pl
from jax.experimental.pallas import tpu as pltpu
from jax.experimental.pallas import tpu_sc as plsc

_L = 200
_D = 128
_B = 1024
_N = _B * _L
_NC, _NS = 2, 16
_NW = _NC * _NS
_PER_W = _N // _NW
_STEP = 128
_NSTEP = _PER_W // _STEP


def _pos_encoding():
    pos = jnp.arange(_L, dtype=jnp.float32)[:, None]
    i = jnp.arange(0, _D, 2, dtype=jnp.float32)
    div = jnp.exp(-jnp.log(10000.0) * i / _D)
    pe = jnp.zeros((_L, _D), dtype=jnp.float32)
    pe = pe.at[:, 0::2].set(jnp.sin(pos * div))
    pe = pe.at[:, 1::2].set(jnp.cos(pos * div))
    return pe


def _embed_body(table_hbm, idx_hbm, pe_hbm, out_hbm,
                idx_v, pe2_v, buf0, buf1, gsem0, gsem1, osem0, osem1):
    wid = lax.axis_index("s") * _NC + lax.axis_index("c")
    base = wid * _PER_W

    pltpu.sync_copy(idx_hbm.at[wid], idx_v)
    pltpu.sync_copy(pe_hbm, pe2_v.at[pl.ds(0, _L)])
    pltpu.sync_copy(pe_hbm, pe2_v.at[pl.ds(_L, _L)])

    bufs = (buf0, buf1)
    gsems = (gsem0, gsem1)
    osems = (osem0, osem1)

    def g_start(j, s):
        pltpu.async_copy(table_hbm.at[idx_v.at[j]], bufs[s], gsems[s])

    def g_wait(j, s):
        pltpu.make_async_copy(table_hbm.at[idx_v.at[j]], bufs[s], gsems[s]).wait()

    def o_start(j, s):
        dst = out_hbm.at[pl.ds(base + j * _STEP, _STEP)]
        pltpu.async_copy(bufs[s], dst, osems[s])

    def o_wait(j, s):
        dst = out_hbm.at[pl.ds(base + j * _STEP, _STEP)]
        pltpu.make_async_copy(bufs[s], dst, osems[s]).wait()

    def add_pe(j, s):
        buf = bufs[s]
        p = lax.rem(j * _STEP, _L)

        @pl.loop(0, _STEP)
        def _row(r):
            pr = p + r
            for c in range(_D // 16):
                sl = pl.ds(c * 16, 16)
                plsc.addupdate(buf.at[r, sl], pe2_v[pr, sl])

    @pl.loop(0, _NSTEP)
    def _step(j):
        g_start(j, 0)
        g_wait(j, 0)
        add_pe(j, 0)
        o_start(j, 0)
        o_wait(j, 0)


def kernel(x, table):
    idx = x.reshape(_NW, _NSTEP, _STEP).astype(jnp.int32)
    pe = _pos_encoding()
    mesh = plsc.VectorSubcoreMesh(core_axis_name="c", subcore_axis_name="s")
    run = pl.kernel(
        _embed_body,
        out_type=jax.ShapeDtypeStruct((_N, _D), jnp.float32),
        mesh=mesh,
        scratch_types=[
            pltpu.VMEM((_NSTEP, _STEP), jnp.int32),
            pltpu.VMEM((2 * _L, _D), jnp.float32),
            pltpu.VMEM((_STEP, _D), jnp.float32),
            pltpu.VMEM((_STEP, _D), jnp.float32),
            pltpu.SemaphoreType.DMA,
            pltpu.SemaphoreType.DMA,
            pltpu.SemaphoreType.DMA,
            pltpu.SemaphoreType.DMA,
        ],
    )
    out = run(table, idx, pe)
    return out.reshape(_B, _L, _D)


# speedup vs baseline: 2.8020x; 1.2929x over previous
import jax
import jax.numpy as jnp
from jax import lax
from jax.experimental import pallas as pl
from jax.experimental.pallas import tpu as pltpu
from jax.experimental.pallas import tpu_sc as plsc

_L = 200
_D = 128
_B = 1024
_N = _B * _L
_NC, _NS = 2, 16
_NW = _NC * _NS
_PER_W = _N // _NW
_STEP = 128
_NSTEP = _PER_W // _STEP


def _pos_encoding():
    pos = jnp.arange(_L, dtype=jnp.float32)[:, None]
    i = jnp.arange(0, _D, 2, dtype=jnp.float32)
    div = jnp.exp(-jnp.log(10000.0) * i / _D)
    pe = jnp.zeros((_L, _D), dtype=jnp.float32)
    pe = pe.at[:, 0::2].set(jnp.sin(pos * div))
    pe = pe.at[:, 1::2].set(jnp.cos(pos * div))
    return pe


def _embed_body(table_hbm, idx_hbm, pe_hbm, out_hbm,
                idx_v, pe2_v, buf0, buf1, gsem0, gsem1, osem0, osem1):
    wid = lax.axis_index("s") * _NC + lax.axis_index("c")
    base = wid * _PER_W

    pltpu.sync_copy(idx_hbm.at[wid], idx_v)
    pltpu.sync_copy(pe_hbm, pe2_v.at[pl.ds(0, _L)])
    pltpu.sync_copy(pe_hbm, pe2_v.at[pl.ds(_L, _L)])

    bufs = (buf0, buf1)
    gsems = (gsem0, gsem1)
    osems = (osem0, osem1)

    def g_start(j, s):
        pltpu.async_copy(table_hbm.at[idx_v.at[j]], bufs[s], gsems[s])

    def g_wait(j, s):
        pltpu.make_async_copy(table_hbm.at[idx_v.at[j]], bufs[s], gsems[s]).wait()

    def o_start(j, s):
        dst = out_hbm.at[pl.ds(base + j * _STEP, _STEP)]
        pltpu.async_copy(bufs[s], dst, osems[s])

    def o_wait(j, s):
        dst = out_hbm.at[pl.ds(base + j * _STEP, _STEP)]
        pltpu.make_async_copy(bufs[s], dst, osems[s]).wait()

    def add_pe(j, s):
        buf = bufs[s]
        p = lax.rem(j * _STEP, _L)

        @pl.loop(0, _STEP)
        def _row(r):
            pr = p + r
            for c in range(_D // 16):
                sl = pl.ds(c * 16, 16)
                plsc.addupdate(buf.at[r, sl], pe2_v[pr, sl])

    g_start(0, 0)

    g_start(1, 1)
    g_wait(0, 0)
    add_pe(0, 0)
    o_start(0, 0)

    @pl.loop(1, _NSTEP - 1, step=2)
    def _pair(jj):
        for s_off in range(2):
            j = jj + s_off
            slot = 1 - s_off
            o_wait(j - 1, 1 - slot)
            g_start(j + 1, 1 - slot)
            g_wait(j, slot)
            add_pe(j, slot)
            o_start(j, slot)

    o_wait(_NSTEP - 2, 0)
    g_wait(_NSTEP - 1, 1)
    add_pe(_NSTEP - 1, 1)
    o_start(_NSTEP - 1, 1)
    o_wait(_NSTEP - 1, 1)


def kernel(x, table):
    idx = x.reshape(_NW, _NSTEP, _STEP).astype(jnp.int32)
    pe = _pos_encoding()
    mesh = plsc.VectorSubcoreMesh(core_axis_name="c", subcore_axis_name="s")
    run = pl.kernel(
        _embed_body,
        out_type=jax.ShapeDtypeStruct((_N, _D), jnp.float32),
        mesh=mesh,
        scratch_types=[
            pltpu.VMEM((_NSTEP, _STEP), jnp.int32),
            pltpu.VMEM((2 * _L, _D), jnp.float32),
            pltpu.VMEM((_STEP, _D), jnp.float32),
            pltpu.VMEM((_STEP, _D), jnp.float32),
            pltpu.SemaphoreType.DMA,
            pltpu.SemaphoreType.DMA,
            pltpu.SemaphoreType.DMA,
            pltpu.SemaphoreType.DMA,
        ],
    )
    out = run(table, idx, pe)
    return out.reshape(_B, _L, _D)


# speedup vs baseline: 6.3310x; 2.2594x over previous
import jax
import jax.numpy as jnp
from jax import lax
from jax.experimental import pallas as pl
from jax.experimental.pallas import tpu as pltpu
from jax.experimental.pallas import tpu_sc as plsc

_L = 200
_D = 128
_B = 1024
_N = _B * _L
_NC, _NS = 2, 16
_NW = _NC * _NS
_PER_W = _N // _NW
_NSTEP = _PER_W // _L
_SPLIT = 128


def _pos_encoding():
    pos = jnp.arange(_L, dtype=jnp.float32)[:, None]
    i = jnp.arange(0, _D, 2, dtype=jnp.float32)
    div = jnp.exp(-jnp.log(10000.0) * i / _D)
    pe = jnp.zeros((_L, _D), dtype=jnp.float32)
    pe = pe.at[:, 0::2].set(jnp.sin(pos * div))
    pe = pe.at[:, 1::2].set(jnp.cos(pos * div))
    return pe


def _embed_body(table_hbm, idx_hbm, pe_hbm, out_hbm,
                idx_v, pe_v, buf0, buf1, gsem0, gsem1, osem0, osem1):
    wid = lax.axis_index("s") * _NC + lax.axis_index("c")
    base = wid * _PER_W

    pltpu.sync_copy(idx_hbm.at[wid], idx_v)
    pltpu.sync_copy(pe_hbm, pe_v)

    bufs = (buf0, buf1)
    gsems = (gsem0, gsem1)
    osems = (osem0, osem1)

    def g_pair(j, s):
        lo = table_hbm.at[idx_v.at[j, pl.ds(0, _SPLIT)]]
        hi = table_hbm.at[idx_v.at[j, pl.ds(_SPLIT, _L - _SPLIT)]]
        return ((lo, bufs[s].at[pl.ds(0, _SPLIT)], gsems[s]),
                (hi, bufs[s].at[pl.ds(_SPLIT, _L - _SPLIT)], gsems[s]))

    def g_start(j, s):
        for src, dst, sem in g_pair(j, s):
            pltpu.async_copy(src, dst, sem)

    def g_wait(j, s):
        for src, dst, sem in g_pair(j, s):
            pltpu.make_async_copy(src, dst, sem).wait()

    def o_start(j, s):
        dst = out_hbm.at[pl.ds(base + j * _L, _L)]
        pltpu.async_copy(bufs[s], dst, osems[s])

    def o_wait(j, s):
        dst = out_hbm.at[pl.ds(base + j * _L, _L)]
        pltpu.make_async_copy(bufs[s], dst, osems[s]).wait()

    def add_pe(s):
        buf = bufs[s]

        @pl.loop(0, _L)
        def _row(r):
            for c in range(_D // 16):
                sl = pl.ds(c * 16, 16)
                plsc.addupdate(buf.at[r, sl], pe_v[r, sl])

    g_start(0, 0)

    g_start(1, 1)
    g_wait(0, 0)
    add_pe(0)
    o_start(0, 0)

    @pl.loop(1, _NSTEP - 1, step=2)
    def _pair(jj):
        for s_off in range(2):
            j = jj + s_off
            slot = 1 - s_off
            o_wait(j - 1, 1 - slot)
            g_start(j + 1, 1 - slot)
            g_wait(j, slot)
            add_pe(slot)
            o_start(j, slot)

    o_wait(_NSTEP - 2, 0)
    g_wait(_NSTEP - 1, 1)
    add_pe(1)
    o_start(_NSTEP - 1, 1)
    o_wait(_NSTEP - 1, 1)


def kernel(x, table):
    idx = x.reshape(_NW, _NSTEP, _L).astype(jnp.int32)
    pe = _pos_encoding()
    mesh = plsc.VectorSubcoreMesh(core_axis_name="c", subcore_axis_name="s")
    run = pl.kernel(
        _embed_body,
        out_type=jax.ShapeDtypeStruct((_N, _D), jnp.float32),
        mesh=mesh,
        scratch_types=[
            pltpu.VMEM((_NSTEP, _L), jnp.int32),
            pltpu.VMEM((_L, _D), jnp.float32),
            pltpu.VMEM((_L, _D), jnp.float32),
            pltpu.VMEM((_L, _D), jnp.float32),
            pltpu.SemaphoreType.DMA,
            pltpu.SemaphoreType.DMA,
            pltpu.SemaphoreType.DMA,
            pltpu.SemaphoreType.DMA,
        ],
    )
    out = run(table, idx, pe)
    return out.reshape(_B, _L, _D)


# speedup vs baseline: 7.3812x; 1.1659x over previous
import numpy as np

import jax
import jax.numpy as jnp
from jax import lax
from jax.experimental import pallas as pl
from jax.experimental.pallas import tpu as pltpu
from jax.experimental.pallas import tpu_sc as plsc

_L = 200
_D = 128
_B = 1024
_N = _B * _L
_NC, _NS = 2, 16
_NW = _NC * _NS
_SENT_W = _B // _NW
_SPLIT = 128
_NBUF = 3


def _pos_encoding_np():
    pos = np.arange(_L, dtype=np.float64)[:, None]
    i = np.arange(0, _D, 2, dtype=np.float64)
    div = np.exp(-np.log(10000.0) * i / _D)
    pe = np.zeros((_L, _D), dtype=np.float32)
    pe[:, 0::2] = np.sin(pos * div).astype(np.float32)
    pe[:, 1::2] = np.cos(pos * div).astype(np.float32)
    return pe


def _embed_body(table_hbm, x_hbm, pe_hbm, out_hbm,
                idx_v, pe_v, buf0, buf1, buf2,
                gsem0, gsem1, gsem2, osem0, osem1, osem2):
    wid = lax.axis_index("s") * _NC + lax.axis_index("c")
    b0 = wid * _SENT_W

    pltpu.sync_copy(x_hbm.at[pl.ds(b0, _SENT_W)], idx_v)
    pltpu.sync_copy(pe_hbm, pe_v)

    bufs = (buf0, buf1, buf2)
    gsems = (gsem0, gsem1, gsem2)
    osems = (osem0, osem1, osem2)

    def g_pair(j, s):
        lo = table_hbm.at[idx_v.at[j, pl.ds(0, _SPLIT)]]
        hi = table_hbm.at[idx_v.at[j, pl.ds(_SPLIT, _L - _SPLIT)]]
        return ((lo, bufs[s].at[pl.ds(0, _SPLIT)], gsems[s]),
                (hi, bufs[s].at[pl.ds(_SPLIT, _L - _SPLIT)], gsems[s]))

    def g_start(j, s):
        for src, dst, sem in g_pair(j, s):
            pltpu.async_copy(src, dst, sem)

    def g_wait(j, s):
        for src, dst, sem in g_pair(j, s):
            pltpu.make_async_copy(src, dst, sem).wait()

    def o_start(j, s):
        pltpu.async_copy(bufs[s], out_hbm.at[b0 + j], osems[s])

    def o_wait(j, s):
        pltpu.make_async_copy(bufs[s], out_hbm.at[b0 + j], osems[s]).wait()

    def add_pe(s):
        buf = bufs[s]

        @pl.loop(0, _L)
        def _row(r):
            for c in range(_D // 16):
                sl = pl.ds(c * 16, 16)
                plsc.addupdate(buf.at[r, sl], pe_v[r, sl])

    def body(j, s, *, wait_out=True, start_next=True):
        if wait_out:
            o_wait(j - 2, (s - 2) % _NBUF)
        if start_next:
            g_start(j + 1, (s + 1) % _NBUF)
        g_wait(j, s)
        add_pe(s)
        o_start(j, s)

    g_start(0, 0)
    body(0, 0, wait_out=False)
    body(1, 1, wait_out=False)

    @pl.loop(2, _SENT_W - 3, step=_NBUF)
    def _trip(jj):
        for off in range(_NBUF):
            body(jj + off, (2 + off) % _NBUF)

    body(_SENT_W - 3, (_SENT_W - 3) % _NBUF)
    body(_SENT_W - 2, (_SENT_W - 2) % _NBUF)
    body(_SENT_W - 1, (_SENT_W - 1) % _NBUF, start_next=False)
    o_wait(_SENT_W - 2, (_SENT_W - 2) % _NBUF)
    o_wait(_SENT_W - 1, (_SENT_W - 1) % _NBUF)


def kernel(x, table):
    pe = jnp.asarray(_pos_encoding_np())
    mesh = plsc.VectorSubcoreMesh(core_axis_name="c", subcore_axis_name="s")
    run = pl.kernel(
        _embed_body,
        out_type=jax.ShapeDtypeStruct((_B, _L, _D), jnp.float32),
        mesh=mesh,
        scratch_types=[
            pltpu.VMEM((_SENT_W, _L), jnp.int32),
            pltpu.VMEM((_L, _D), jnp.float32),
            pltpu.VMEM((_L, _D), jnp.float32),
            pltpu.VMEM((_L, _D), jnp.float32),
            pltpu.VMEM((_L, _D), jnp.float32),
            pltpu.SemaphoreType.DMA,
            pltpu.SemaphoreType.DMA,
            pltpu.SemaphoreType.DMA,
            pltpu.SemaphoreType.DMA,
            pltpu.SemaphoreType.DMA,
            pltpu.SemaphoreType.DMA,
        ],
    )
    return run(table, x.astype(jnp.int32), pe)
